# Initial kernel scaffold; baseline (speedup 1.0000x reference)
import jax
import jax.numpy as jnp
from jax.experimental import pallas as pl


def kernel(classifications, regressions, anchors, annotations, dataset):
    raise NotImplementedError("write your pallas kernel here")


# speedup vs baseline: 2.9457x; 2.9457x over previous
import functools

import jax
import jax.numpy as jnp
from jax.experimental import pallas as pl

_B = 8
_A = 49104
_C = 80
_MAX_ANN = 20
_BLK = 5456
_NB = _A // _BLK


def _loss_kernel(cls_ref, reg_ref, anc_ref, ann_ref, out_ref):
    i = pl.program_id(1)

    ann = ann_ref[0]
    bx1 = ann[0:1, :]
    by1 = ann[1:2, :]
    bx2 = ann[2:3, :]
    by2 = ann[3:4, :]
    blab = ann[4:5, :]
    valid = blab != -1.0
    has_ann = jnp.any(valid)

    anc = anc_ref[0]
    ax1 = anc[:, 0:1]
    ay1 = anc[:, 1:2]
    ax2 = anc[:, 2:3]
    ay2 = anc[:, 3:4]
    aw = ax2 - ax1
    ah = ay2 - ay1
    acx = ax1 + 0.5 * aw
    acy = ay1 + 0.5 * ah

    iw = jnp.clip(jnp.minimum(ax2, bx2) - jnp.maximum(ax1, bx1), 0.0, None)
    ih = jnp.clip(jnp.minimum(ay2, by2) - jnp.maximum(ay1, by1), 0.0, None)
    inter = iw * ih
    area_b = (bx2 - bx1) * (by2 - by1)
    ua = jnp.clip(aw * ah + area_b - inter, 1e-8, None)
    iou = jnp.where(valid, inter / ua, -1.0)

    iou_max = jnp.max(iou, axis=1, keepdims=True)
    col = jax.lax.broadcasted_iota(jnp.int32, iou.shape, 1)
    cand = jnp.where(iou == iou_max, col, _MAX_ANN)
    amin = jnp.min(cand, axis=1, keepdims=True)
    onehot = (col == amin).astype(jnp.float32)

    assigned = jax.lax.dot_general(
        onehot, ann, (((1,), (1,)), ((), ())),
        preferred_element_type=jnp.float32)

    positive = (iou_max >= 0.5) & has_ann
    contrib = positive | (iou_max < 0.4)

    c = jnp.clip(cls_ref[0], 1e-4, 1.0 - 1e-4)
    neg_elem = 0.75 * c * c * (-jnp.log(1.0 - c))
    pos_elem = 0.25 * (1.0 - c) * (1.0 - c) * (-jnp.log(c))
    base = jnp.where(contrib, neg_elem, 0.0)
    lab = (assigned[:, 4:5] + 0.5).astype(jnp.int32)
    ccol = jax.lax.broadcasted_iota(jnp.int32, c.shape, 1)
    sel = positive & (ccol == lab)
    corr = jnp.where(sel, pos_elem - neg_elem, 0.0)
    cls_partial = jnp.sum(base) + jnp.sum(corr)

    num_pos = jnp.sum(positive.astype(jnp.float32))

    g0 = assigned[:, 0:1]
    g1 = assigned[:, 1:2]
    g2 = assigned[:, 2:3]
    g3 = assigned[:, 3:4]
    gwr = g2 - g0
    ghr = g3 - g1
    gcx = g0 + 0.5 * gwr
    gcy = g1 + 0.5 * ghr
    gw = jnp.clip(gwr, 1.0, None)
    gh = jnp.clip(ghr, 1.0, None)
    t0 = (gcx - acx) / aw * (1.0 / 0.1)
    t1 = (gcy - acy) / ah * (1.0 / 0.1)
    t2 = jnp.log(gw / aw) * (1.0 / 0.2)
    t3 = jnp.log(gh / ah) * (1.0 / 0.2)
    reg_t = jnp.concatenate([t0, t1, t2, t3], axis=1)
    diff = jnp.abs(reg_t - reg_ref[0])
    rl = jnp.where(diff <= 1.0 / 9.0, 4.5 * diff * diff, diff - 0.5 / 9.0)
    reg_partial = jnp.sum(jnp.where(positive, rl, 0.0))

    lane = jax.lax.broadcasted_iota(jnp.int32, (1, 1, 128), 2)
    vec = (jnp.where(lane == 0, cls_partial, 0.0)
           + jnp.where(lane == 1, reg_partial, 0.0)
           + jnp.where(lane == 2, num_pos, 0.0))

    @pl.when(i == 0)
    def _init():
        out_ref[...] = vec

    @pl.when(i != 0)
    def _acc():
        out_ref[...] += vec


@jax.jit
def _run(classifications, regressions, anchors, annotations):
    ann_t = jnp.transpose(annotations, (0, 2, 1))
    sums = pl.pallas_call(
        _loss_kernel,
        grid=(_B, _NB),
        in_specs=[
            pl.BlockSpec((1, _BLK, _C), lambda b, i: (b, i, 0)),
            pl.BlockSpec((1, _BLK, 4), lambda b, i: (b, i, 0)),
            pl.BlockSpec((1, _BLK, 4), lambda b, i: (0, i, 0)),
            pl.BlockSpec((1, 5, _MAX_ANN), lambda b, i: (b, 0, 0)),
        ],
        out_specs=pl.BlockSpec((1, 1, 128), lambda b, i: (b, 0, 0)),
        out_shape=jax.ShapeDtypeStruct((_B, 1, 128), jnp.float32),
    )(classifications, regressions, anchors, ann_t)
    cls_sum = sums[:, 0, 0]
    reg_sum = sums[:, 0, 1]
    num_pos = sums[:, 0, 2]
    cls_total = cls_sum / jnp.clip(num_pos, 1.0, None)
    reg_total = jnp.where(num_pos > 0,
                          reg_sum / jnp.clip(num_pos * 4.0, 1.0, None), 0.0)
    return jnp.stack([cls_total.mean(), reg_total.mean()])


def kernel(classifications, regressions, anchors, annotations, dataset=0):
    return _run(classifications, regressions, anchors, annotations)
